# Initial kernel scaffold; baseline (speedup 1.0000x reference)
import jax
import jax.numpy as jnp
from jax.experimental import pallas as pl


def kernel(x, anchor):
    raise NotImplementedError("write your pallas kernel here")


# speedup vs baseline: 19.9270x; 19.9270x over previous
import jax
import jax.numpy as jnp
from jax.experimental import pallas as pl

_B = 8
_N = 5000
_CH = 85
_NPAD = 5120
_TOPK = 200
_IOU_T = 0.5
_SCORE_T = 0.01


def _decode_body(xt_ref, at_ref, s_ref, c_ref, x1_ref, y1_ref, x2_ref, y2_ref):
    l = xt_ref[0]
    r = jax.lax.broadcasted_iota(jnp.int32, (_CH, _NPAD), 0)
    cls_row = r >= 4
    lc = jnp.where(cls_row, l, -1e30)
    m = jnp.max(lc, axis=0, keepdims=True)
    e = jnp.where(cls_row, jnp.exp(l - m), 0.0)
    s_sum = jnp.sum(e, axis=0, keepdims=True)
    cp = e / s_sum
    cp1 = jnp.where(r >= 5, cp, -1.0)
    sc = jnp.max(cp1, axis=0, keepdims=True)
    s_ref[0] = sc
    ridx = jnp.where(cp1 == sc, r, 1 << 30)
    c_ref[0] = (jnp.min(ridx, axis=0, keepdims=True) - 5).astype(jnp.float32)

    d_x = l[0:1, :]
    d_y = l[1:2, :]
    d_w = l[2:3, :]
    d_h = l[3:4, :]
    a_x = at_ref[0:1, :]
    a_y = at_ref[1:2, :]
    a_w = at_ref[2:3, :]
    a_h = at_ref[3:4, :]
    cx = d_x * a_w / 10.0 + a_x
    cy = d_y * a_h / 10.0 + a_y
    w = jnp.exp(d_w / 5.0) * a_w
    h = jnp.exp(d_h / 5.0) * a_h
    x1_ref[0] = cx - w / 2.0
    y1_ref[0] = cy - h / 2.0
    x2_ref[0] = cx + w / 2.0
    y2_ref[0] = cy + h / 2.0


def _nms_body(s_ref, c_ref, x1_ref, y1_ref, x2_ref, y2_ref, out_ref):
    lane = jax.lax.broadcasted_iota(jnp.int32, (_B, _NPAD), 1)
    sa0 = jnp.where(lane < _N, s_ref[...], -2.0)
    cv = c_ref[...]
    x1 = x1_ref[...]
    y1 = y1_ref[...]
    x2 = x2_ref[...]
    y2 = y2_ref[...]
    area = (x2 - x1) * (y2 - y1)
    k = jax.lax.broadcasted_iota(jnp.int32, (_B, 8), 1)

    def body(i, sa):
        m = jnp.max(sa, axis=1, keepdims=True)
        found = m >= _SCORE_T
        idx = jnp.min(jnp.where(sa == m, lane, 1 << 30), axis=1, keepdims=True)
        sel = lane == idx

        def pick(v):
            return jnp.sum(jnp.where(sel, v, 0.0), axis=1, keepdims=True)

        bx1 = pick(x1)
        by1 = pick(y1)
        bx2 = pick(x2)
        by2 = pick(y2)
        bc = pick(cv)
        a1 = (bx2 - bx1) * (by2 - by1)
        xl = jnp.maximum(bx1, x1)
        xr = jnp.minimum(bx2, x2)
        yt = jnp.maximum(by1, y1)
        yb = jnp.minimum(by2, y2)
        common = jnp.clip(xr - xl, 0.0, 1.0) * jnp.clip(yb - yt, 0.0, 1.0)
        iou = common / (a1 + area - common)
        supp = (iou >= _IOU_T) | sel
        sa = jnp.where(found & supp, -2.0, sa)

        vals = jnp.where(k == 0, bc, 0.0)
        vals = jnp.where(k == 1, m, vals)
        vals = jnp.where(k == 2, bx1, vals)
        vals = jnp.where(k == 3, by1, vals)
        vals = jnp.where(k == 4, bx2, vals)
        vals = jnp.where(k == 5, by2, vals)
        vals = jnp.where(found, vals, 0.0)
        out_ref[pl.ds(i, 1), :, :] = vals[None, :, :]
        return sa

    jax.lax.fori_loop(0, _TOPK, body, sa0, unroll=False)


def kernel(x, anchor):
    xt = jnp.transpose(x, (0, 2, 1))
    xt = jnp.pad(xt, ((0, 0), (0, 0), (0, _NPAD - _N)))
    at = jnp.pad(anchor.T, ((0, 0), (0, _NPAD - _N)))

    row = jax.ShapeDtypeStruct((_B, 1, _NPAD), jnp.float32)
    outs = pl.pallas_call(
        _decode_body,
        grid=(_B,),
        in_specs=[
            pl.BlockSpec((1, _CH, _NPAD), lambda b: (b, 0, 0)),
            pl.BlockSpec((4, _NPAD), lambda b: (0, 0)),
        ],
        out_specs=[pl.BlockSpec((1, 1, _NPAD), lambda b: (b, 0, 0))] * 6,
        out_shape=[row] * 6,
    )(xt, at)
    s, c, x1, y1, x2, y2 = (o.reshape(_B, _NPAD) for o in outs)

    out = pl.pallas_call(
        _nms_body,
        out_shape=jax.ShapeDtypeStruct((_TOPK, _B, 8), jnp.float32),
    )(s, c, x1, y1, x2, y2)
    return jnp.transpose(out, (1, 0, 2))[:, :, :6]


# speedup vs baseline: 21.6123x; 1.0846x over previous
import jax
import jax.numpy as jnp
from jax.experimental import pallas as pl

_B = 8
_N = 5000
_CH = 85
_TOPK = 200
_IOU_T = 0.5
_SCORE_T = 0.01


def _decode_body(xt_ref, at_ref, s_ref, c_ref, x1_ref, y1_ref, x2_ref, y2_ref):
    l = xt_ref[0]
    r = jax.lax.broadcasted_iota(jnp.int32, (_CH, _N), 0)
    cls_row = r >= 4
    lc = jnp.where(cls_row, l, -1e30)
    m = jnp.max(lc, axis=0, keepdims=True)
    e = jnp.where(cls_row, jnp.exp(l - m), 0.0)
    s_sum = jnp.sum(e, axis=0, keepdims=True)
    cp = e / s_sum
    cp1 = jnp.where(r >= 5, cp, -1.0)
    sc = jnp.max(cp1, axis=0, keepdims=True)
    s_ref[0] = sc
    ridx = jnp.where(cp1 == sc, r, 1 << 30)
    c_ref[0] = (jnp.min(ridx, axis=0, keepdims=True) - 5).astype(jnp.float32)

    d_x = l[0:1, :]
    d_y = l[1:2, :]
    d_w = l[2:3, :]
    d_h = l[3:4, :]
    a_x = at_ref[0:1, :]
    a_y = at_ref[1:2, :]
    a_w = at_ref[2:3, :]
    a_h = at_ref[3:4, :]
    cx = d_x * a_w / 10.0 + a_x
    cy = d_y * a_h / 10.0 + a_y
    w = jnp.exp(d_w / 5.0) * a_w
    h = jnp.exp(d_h / 5.0) * a_h
    x1_ref[0] = cx - w / 2.0
    y1_ref[0] = cy - h / 2.0
    x2_ref[0] = cx + w / 2.0
    y2_ref[0] = cy + h / 2.0


def _nms_body(f_ref, out_ref):
    sa0 = f_ref[0]
    cv = f_ref[1]
    x1 = f_ref[2]
    y1 = f_ref[3]
    x2 = f_ref[4]
    y2 = f_ref[5]
    lane = jax.lax.broadcasted_iota(jnp.int32, (_B, _N), 1)
    area = (x2 - x1) * (y2 - y1)
    k = jax.lax.broadcasted_iota(jnp.int32, (_B, 8), 1)

    def body(i, sa):
        m = jnp.max(sa, axis=1, keepdims=True)
        found = m >= _SCORE_T
        idx = jnp.min(jnp.where(sa == m, lane, 1 << 30), axis=1, keepdims=True)
        sel = lane == idx

        def pick(v):
            return jnp.sum(jnp.where(sel, v, 0.0), axis=1, keepdims=True)

        bx1 = pick(x1)
        by1 = pick(y1)
        bx2 = pick(x2)
        by2 = pick(y2)
        bc = pick(cv)
        a1 = (bx2 - bx1) * (by2 - by1)
        xl = jnp.maximum(bx1, x1)
        xr = jnp.minimum(bx2, x2)
        yt = jnp.maximum(by1, y1)
        yb = jnp.minimum(by2, y2)
        common = jnp.clip(xr - xl, 0.0, 1.0) * jnp.clip(yb - yt, 0.0, 1.0)
        iou = common / (a1 + area - common)
        sa = jnp.where((iou >= _IOU_T) | sel, -2.0, sa)

        vals = jnp.where(k == 0, bc, 0.0)
        vals = jnp.where(k == 1, m, vals)
        vals = jnp.where(k == 2, bx1, vals)
        vals = jnp.where(k == 3, by1, vals)
        vals = jnp.where(k == 4, bx2, vals)
        vals = jnp.where(k == 5, by2, vals)
        vals = jnp.where(found, vals, 0.0)
        out_ref[pl.ds(i, 1), :, :] = vals[None, :, :]
        return sa

    jax.lax.fori_loop(0, _TOPK, body, sa0, unroll=False)


def kernel(x, anchor):
    xt = jnp.transpose(x, (0, 2, 1))
    at = anchor.T

    row = jax.ShapeDtypeStruct((_B, 1, _N), jnp.float32)
    outs = pl.pallas_call(
        _decode_body,
        grid=(_B,),
        in_specs=[
            pl.BlockSpec((1, _CH, _N), lambda b: (b, 0, 0)),
            pl.BlockSpec((4, _N), lambda b: (0, 0)),
        ],
        out_specs=[pl.BlockSpec((1, 1, _N), lambda b: (b, 0, 0))] * 6,
        out_shape=[row] * 6,
    )(xt, at)
    fields = jnp.concatenate(outs, axis=1).transpose(1, 0, 2)

    out = pl.pallas_call(
        _nms_body,
        out_shape=jax.ShapeDtypeStruct((_TOPK, _B, 8), jnp.float32),
    )(fields)
    return jnp.transpose(out, (1, 0, 2))[:, :, :6]


# speedup vs baseline: 21.9203x; 1.0143x over previous
import jax
import jax.numpy as jnp
from jax.experimental import pallas as pl

_B = 8
_N = 5000
_CH = 85
_TOPK = 200
_IOU_T = 0.5
_SCORE_T = 0.01


def _decode_body(xt_ref, at_ref, s_ref, c_ref, x1_ref, y1_ref, x2_ref, y2_ref):
    l = xt_ref[0]
    r = jax.lax.broadcasted_iota(jnp.int32, (_CH, _N), 0)
    cls_row = r >= 4
    lc = jnp.where(cls_row, l, -1e30)
    m = jnp.max(lc, axis=0, keepdims=True)
    e = jnp.where(cls_row, jnp.exp(l - m), 0.0)
    s_sum = jnp.sum(e, axis=0, keepdims=True)
    cp = e / s_sum
    cp1 = jnp.where(r >= 5, cp, -1.0)
    sc = jnp.max(cp1, axis=0, keepdims=True)
    s_ref[0] = sc
    ridx = jnp.where(cp1 == sc, r, 1 << 30)
    c_ref[0] = (jnp.min(ridx, axis=0, keepdims=True) - 5).astype(jnp.float32)

    d_x = l[0:1, :]
    d_y = l[1:2, :]
    d_w = l[2:3, :]
    d_h = l[3:4, :]
    a_x = at_ref[0:1, :]
    a_y = at_ref[1:2, :]
    a_w = at_ref[2:3, :]
    a_h = at_ref[3:4, :]
    cx = d_x * a_w / 10.0 + a_x
    cy = d_y * a_h / 10.0 + a_y
    w = jnp.exp(d_w / 5.0) * a_w
    h = jnp.exp(d_h / 5.0) * a_h
    x1_ref[0] = cx - w / 2.0
    y1_ref[0] = cy - h / 2.0
    x2_ref[0] = cx + w / 2.0
    y2_ref[0] = cy + h / 2.0


def _nms_body(f_ref, out_ref):
    sa0 = f_ref[0]
    cv = f_ref[1]
    x1 = f_ref[2]
    y1 = f_ref[3]
    x2 = f_ref[4]
    y2 = f_ref[5]
    lane = jax.lax.broadcasted_iota(jnp.int32, (_B, _N), 1)
    area = (x2 - x1) * (y2 - y1)
    k = jax.lax.broadcasted_iota(jnp.int32, (_B, 8), 1)

    def body(i, sa):
        m = jnp.max(sa, axis=1, keepdims=True)
        found = m >= _SCORE_T
        idx = jnp.min(jnp.where(sa == m, lane, 1 << 30), axis=1, keepdims=True)
        sel = lane == idx

        def pick(v):
            return jnp.sum(jnp.where(sel, v, 0.0), axis=1, keepdims=True)

        bx1 = pick(x1)
        by1 = pick(y1)
        bx2 = pick(x2)
        by2 = pick(y2)
        bc = pick(cv)
        a1 = (bx2 - bx1) * (by2 - by1)
        xl = jnp.maximum(bx1, x1)
        xr = jnp.minimum(bx2, x2)
        yt = jnp.maximum(by1, y1)
        yb = jnp.minimum(by2, y2)
        common = jnp.clip(xr - xl, 0.0, 1.0) * jnp.clip(yb - yt, 0.0, 1.0)
        iou = common / (a1 + area - common)
        sa = jnp.where((iou >= _IOU_T) | sel, -2.0, sa)

        vals = jnp.where(k == 0, bc, 0.0)
        vals = jnp.where(k == 1, m, vals)
        vals = jnp.where(k == 2, bx1, vals)
        vals = jnp.where(k == 3, by1, vals)
        vals = jnp.where(k == 4, bx2, vals)
        vals = jnp.where(k == 5, by2, vals)
        vals = jnp.where(found, vals, 0.0)
        out_ref[pl.ds(i, 1), :, :] = vals[None, :, :]
        return sa

    jax.lax.fori_loop(0, _TOPK, body, sa0, unroll=2)


def kernel(x, anchor):
    xt = jnp.transpose(x, (0, 2, 1))
    at = anchor.T

    row = jax.ShapeDtypeStruct((_B, 1, _N), jnp.float32)
    outs = pl.pallas_call(
        _decode_body,
        grid=(_B,),
        in_specs=[
            pl.BlockSpec((1, _CH, _N), lambda b: (b, 0, 0)),
            pl.BlockSpec((4, _N), lambda b: (0, 0)),
        ],
        out_specs=[pl.BlockSpec((1, 1, _N), lambda b: (b, 0, 0))] * 6,
        out_shape=[row] * 6,
    )(xt, at)
    fields = jnp.concatenate(outs, axis=1).transpose(1, 0, 2)

    out = pl.pallas_call(
        _nms_body,
        out_shape=jax.ShapeDtypeStruct((_TOPK, _B, 8), jnp.float32),
    )(fields)
    return jnp.transpose(out, (1, 0, 2))[:, :, :6]


# speedup vs baseline: 24.2255x; 1.1052x over previous
import jax
import jax.numpy as jnp
from jax.experimental import pallas as pl

_B = 8
_N = 5000
_TOPK = 200
_IOU_T = 0.5
_SCORE_T = 0.01


def _decode_body(xt_ref, at_ref, x1_ref, y1_ref, x2_ref, y2_ref):
    d = xt_ref[0]
    d_x = d[0:1, :]
    d_y = d[1:2, :]
    d_w = d[2:3, :]
    d_h = d[3:4, :]
    a_x = at_ref[0:1, :]
    a_y = at_ref[1:2, :]
    a_w = at_ref[2:3, :]
    a_h = at_ref[3:4, :]
    cx = d_x * a_w / 10.0 + a_x
    cy = d_y * a_h / 10.0 + a_y
    w = jnp.exp(d_w / 5.0) * a_w
    h = jnp.exp(d_h / 5.0) * a_h
    x1_ref[0] = cx - w / 2.0
    y1_ref[0] = cy - h / 2.0
    x2_ref[0] = cx + w / 2.0
    y2_ref[0] = cy + h / 2.0


def _nms_body(f_ref, out_ref):
    sa0 = f_ref[0]
    cv = f_ref[1]
    x1 = f_ref[2]
    y1 = f_ref[3]
    x2 = f_ref[4]
    y2 = f_ref[5]
    lane = jax.lax.broadcasted_iota(jnp.int32, (_B, _N), 1)
    area = (x2 - x1) * (y2 - y1)
    k = jax.lax.broadcasted_iota(jnp.int32, (_B, 8), 1)

    def body(i, sa):
        m = jnp.max(sa, axis=1, keepdims=True)
        found = m >= _SCORE_T
        idx = jnp.min(jnp.where(sa == m, lane, 1 << 30), axis=1, keepdims=True)
        sel = lane == idx

        def pick(v):
            return jnp.sum(jnp.where(sel, v, 0.0), axis=1, keepdims=True)

        bx1 = pick(x1)
        by1 = pick(y1)
        bx2 = pick(x2)
        by2 = pick(y2)
        bc = pick(cv)
        a1 = (bx2 - bx1) * (by2 - by1)
        xl = jnp.maximum(bx1, x1)
        xr = jnp.minimum(bx2, x2)
        yt = jnp.maximum(by1, y1)
        yb = jnp.minimum(by2, y2)
        common = jnp.clip(xr - xl, 0.0, 1.0) * jnp.clip(yb - yt, 0.0, 1.0)
        iou = common / (a1 + area - common)
        sa = jnp.where((iou >= _IOU_T) | sel, -2.0, sa)

        vals = jnp.where(k == 0, bc, 0.0)
        vals = jnp.where(k == 1, m, vals)
        vals = jnp.where(k == 2, bx1, vals)
        vals = jnp.where(k == 3, by1, vals)
        vals = jnp.where(k == 4, bx2, vals)
        vals = jnp.where(k == 5, by2, vals)
        vals = jnp.where(found, vals, 0.0)
        out_ref[pl.ds(i, 1), :, :] = vals[None, :, :]
        return sa

    jax.lax.fori_loop(0, _TOPK, body, sa0, unroll=2)


def kernel(x, anchor):
    cp = jax.nn.softmax(x[:, :, 4:], axis=2)
    s = jnp.max(cp[:, :, 1:], axis=2)
    c = jnp.argmax(cp[:, :, 1:], axis=2).astype(jnp.float32)

    xt = jnp.transpose(x[:, :, :4], (0, 2, 1))
    at = anchor.T

    row = jax.ShapeDtypeStruct((_B, 1, _N), jnp.float32)
    boxes = pl.pallas_call(
        _decode_body,
        grid=(_B,),
        in_specs=[
            pl.BlockSpec((1, 4, _N), lambda b: (b, 0, 0)),
            pl.BlockSpec((4, _N), lambda b: (0, 0)),
        ],
        out_specs=[pl.BlockSpec((1, 1, _N), lambda b: (b, 0, 0))] * 4,
        out_shape=[row] * 4,
    )(xt, at)

    fields = jnp.stack(
        [s, c] + [b.reshape(_B, _N) for b in boxes], axis=0
    )

    out = pl.pallas_call(
        _nms_body,
        out_shape=jax.ShapeDtypeStruct((_TOPK, _B, 8), jnp.float32),
    )(fields)
    return jnp.transpose(out, (1, 0, 2))[:, :, :6]
